# initial kernel scaffold (unmeasured)
import jax
import jax.numpy as jnp
from jax import lax
from jax.experimental import pallas as pl
from jax.experimental.pallas import tpu as pltpu


def kernel(
    x,
):
    def body(*refs):
        pass

    out_shape = jax.ShapeDtypeStruct(..., jnp.float32)
    return pl.pallas_call(body, out_shape=out_shape)(...)


# baseline (device time: 10670 ns/iter reference)
import jax
import jax.numpy as jnp
from jax import lax
from jax.experimental import pallas as pl
from jax.experimental.pallas import tpu as pltpu


def kernel(x):
    m, n = x.shape

    def body(x_ref, out_ref, send_buf, peer_buf, red_buf, gat_buf,
             send_sems, recv_sems):
        my_x = lax.axis_index("x")
        my_y = lax.axis_index("y")
        x_peer = (1 - my_x, my_y)
        y_peer = (my_x, 1 - my_y)

        barrier_sem = pltpu.get_barrier_semaphore()
        for nbr in (x_peer, y_peer):
            pl.semaphore_signal(
                barrier_sem, inc=1,
                device_id=nbr, device_id_type=pl.DeviceIdType.MESH,
            )
        pl.semaphore_wait(barrier_sem, 2)

        send_buf[...] = x_ref[...].astype(jnp.bfloat16)
        rdma1 = pltpu.make_async_remote_copy(
            src_ref=send_buf,
            dst_ref=peer_buf,
            send_sem=send_sems.at[0],
            recv_sem=recv_sems.at[0],
            device_id=x_peer,
            device_id_type=pl.DeviceIdType.MESH,
        )
        rdma1.start()
        rdma1.wait()

        red_buf[...] = send_buf[...] + peer_buf[...]

        rdma2 = pltpu.make_async_remote_copy(
            src_ref=red_buf,
            dst_ref=gat_buf,
            send_sem=send_sems.at[1],
            recv_sem=recv_sems.at[1],
            device_id=y_peer,
            device_id_type=pl.DeviceIdType.MESH,
        )
        rdma2.start()
        rdma2.wait()

        @pl.when(my_y == 0)
        def _():
            out_ref[:, :n] = red_buf[...].astype(jnp.float32)
            out_ref[:, n:] = gat_buf[...].astype(jnp.float32)

        @pl.when(my_y == 1)
        def _():
            out_ref[:, :n] = gat_buf[...].astype(jnp.float32)
            out_ref[:, n:] = red_buf[...].astype(jnp.float32)

    return pl.pallas_call(
        body,
        out_shape=jax.ShapeDtypeStruct((m, 2 * n), jnp.float32),
        in_specs=[pl.BlockSpec(memory_space=pltpu.VMEM)],
        out_specs=pl.BlockSpec(memory_space=pltpu.VMEM),
        scratch_shapes=[
            pltpu.VMEM((m, n), jnp.bfloat16),
            pltpu.VMEM((m, n), jnp.bfloat16),
            pltpu.VMEM((m, n), jnp.bfloat16),
            pltpu.VMEM((m, n), jnp.bfloat16),
            pltpu.SemaphoreType.DMA((2,)),
            pltpu.SemaphoreType.DMA((2,)),
        ],
        compiler_params=pltpu.CompilerParams(collective_id=0),
    )(x)


# device time: 9713 ns/iter; 1.0985x vs baseline; 1.0985x over previous
import jax
import jax.numpy as jnp
from jax import lax
from jax.experimental import pallas as pl
from jax.experimental.pallas import tpu as pltpu

NCHUNK = 4


def kernel(x):
    m, n = x.shape
    ck = m // NCHUNK

    def body(x_ref, out_ref, send_buf, peer_buf, red_buf, gat_buf,
             p1_send, p1_recv, p2_send, p2_recv):
        my_x = lax.axis_index("x")
        my_y = lax.axis_index("y")
        x_peer = (1 - my_x, my_y)
        y_peer = (my_x, 1 - my_y)

        barrier_sem = pltpu.get_barrier_semaphore()
        for nbr in (x_peer, y_peer):
            pl.semaphore_signal(
                barrier_sem, inc=1,
                device_id=nbr, device_id_type=pl.DeviceIdType.MESH,
            )
        pl.semaphore_wait(barrier_sem, 2)

        send_buf[...] = x_ref[...].astype(jnp.bfloat16)

        def p1_rdma(k):
            sl = pl.ds(k * ck, ck)
            return pltpu.make_async_remote_copy(
                src_ref=send_buf.at[sl],
                dst_ref=peer_buf.at[sl],
                send_sem=p1_send.at[k],
                recv_sem=p1_recv.at[k],
                device_id=x_peer,
                device_id_type=pl.DeviceIdType.MESH,
            )

        def p2_rdma(k):
            sl = pl.ds(k * ck, ck)
            return pltpu.make_async_remote_copy(
                src_ref=red_buf.at[sl],
                dst_ref=gat_buf.at[sl],
                send_sem=p2_send.at[k],
                recv_sem=p2_recv.at[k],
                device_id=y_peer,
                device_id_type=pl.DeviceIdType.MESH,
            )

        p1 = [p1_rdma(k) for k in range(NCHUNK)]
        for k in range(NCHUNK):
            p1[k].start()

        my_col = my_y * n
        other_col = (1 - my_y) * n

        p2 = [p2_rdma(k) for k in range(NCHUNK)]
        for k in range(NCHUNK):
            sl = pl.ds(k * ck, ck)
            p1[k].wait()
            red_buf[sl] = send_buf[sl] + peer_buf[sl]
            p2[k].start()
            out_ref[sl, pl.ds(my_col, n)] = red_buf[sl].astype(jnp.float32)

        for k in range(NCHUNK):
            sl = pl.ds(k * ck, ck)
            p2[k].wait()
            out_ref[sl, pl.ds(other_col, n)] = gat_buf[sl].astype(jnp.float32)

    return pl.pallas_call(
        body,
        out_shape=jax.ShapeDtypeStruct((m, 2 * n), jnp.float32),
        in_specs=[pl.BlockSpec(memory_space=pltpu.VMEM)],
        out_specs=pl.BlockSpec(memory_space=pltpu.VMEM),
        scratch_shapes=[
            pltpu.VMEM((m, n), jnp.bfloat16),
            pltpu.VMEM((m, n), jnp.bfloat16),
            pltpu.VMEM((m, n), jnp.bfloat16),
            pltpu.VMEM((m, n), jnp.bfloat16),
            pltpu.SemaphoreType.DMA((NCHUNK,)),
            pltpu.SemaphoreType.DMA((NCHUNK,)),
            pltpu.SemaphoreType.DMA((NCHUNK,)),
            pltpu.SemaphoreType.DMA((NCHUNK,)),
        ],
        compiler_params=pltpu.CompilerParams(collective_id=0),
    )(x)


# device time: 9583 ns/iter; 1.1134x vs baseline; 1.0136x over previous
import jax
import jax.numpy as jnp
from jax import lax
from jax.experimental import pallas as pl
from jax.experimental.pallas import tpu as pltpu

NCHUNK = 4


def kernel(x):
    m, n = x.shape
    ck = m // NCHUNK

    def body(x_ref, out_ref, send_buf, peer_buf, p1_send, p1_recv,
             p2_send, p2_recv):
        my_x = lax.axis_index("x")
        my_y = lax.axis_index("y")
        x_peer = (1 - my_x, my_y)
        y_peer = (my_x, 1 - my_y)

        barrier_sem = pltpu.get_barrier_semaphore()
        for nbr in (x_peer, y_peer):
            pl.semaphore_signal(
                barrier_sem, inc=1,
                device_id=nbr, device_id_type=pl.DeviceIdType.MESH,
            )
        pl.semaphore_wait(barrier_sem, 2)

        send_buf[...] = x_ref[...].astype(jnp.bfloat16)

        my_col = pl.ds(my_y * n, n)

        def p1_rdma(k):
            sl = pl.ds(k * ck, ck)
            return pltpu.make_async_remote_copy(
                src_ref=send_buf.at[sl],
                dst_ref=peer_buf.at[sl],
                send_sem=p1_send.at[k],
                recv_sem=p1_recv.at[k],
                device_id=x_peer,
                device_id_type=pl.DeviceIdType.MESH,
            )

        def p2_rdma(k):
            sl = pl.ds(k * ck, ck)
            return pltpu.make_async_remote_copy(
                src_ref=out_ref.at[sl, my_col],
                dst_ref=out_ref.at[sl, my_col],
                send_sem=p2_send.at[k],
                recv_sem=p2_recv.at[k],
                device_id=y_peer,
                device_id_type=pl.DeviceIdType.MESH,
            )

        p1 = [p1_rdma(k) for k in range(NCHUNK)]
        for k in range(NCHUNK):
            p1[k].start()

        p2 = [p2_rdma(k) for k in range(NCHUNK)]
        for k in range(NCHUNK):
            sl = pl.ds(k * ck, ck)
            p1[k].wait()
            out_ref[sl, my_col] = send_buf[sl] + peer_buf[sl]
            p2[k].start()

        for k in range(NCHUNK):
            p2[k].wait()

    return pl.pallas_call(
        body,
        out_shape=jax.ShapeDtypeStruct((m, 2 * n), jnp.bfloat16),
        in_specs=[pl.BlockSpec(memory_space=pltpu.VMEM)],
        out_specs=pl.BlockSpec(memory_space=pltpu.VMEM),
        scratch_shapes=[
            pltpu.VMEM((m, n), jnp.bfloat16),
            pltpu.VMEM((m, n), jnp.bfloat16),
            pltpu.SemaphoreType.DMA((NCHUNK,)),
            pltpu.SemaphoreType.DMA((NCHUNK,)),
            pltpu.SemaphoreType.DMA((NCHUNK,)),
            pltpu.SemaphoreType.DMA((NCHUNK,)),
        ],
        compiler_params=pltpu.CompilerParams(collective_id=0),
    )(x)
